# baseline (speedup 1.0000x reference)
import functools

import jax
import jax.numpy as jnp
from jax import lax
from jax.experimental import pallas as pl
from jax.experimental.pallas import tpu as pltpu
from jax.experimental.pallas import tpu_sc as plsc

F32 = jnp.float32
BF16 = jnp.bfloat16
NG = 10000
NM = 10000
NE = 320000
NEH = NE // 2
H = 128

NC = 2
NS = 16
NW = NC * NS

EPW = NEH // NW
CHG = 100
KG = 2
SRG = KG * CHG
NCHUNKG = EPW // CHG
NSUPG = EPW // SRG
CHS = 40
NSUPS = EPW // CHS
NGP = 10240
ACC_PW = NGP // NS
ZCH = 64
EBLK = 1280


def _mesh():
  return plsc.VectorSubcoreMesh(core_axis_name="c", subcore_axis_name="s",
                                num_cores=NC, num_subcores=NS)


def _proj_body(mesh_ref, grid_ref, w1b_ref, w1c_ref, be1_ref,
               psrc_ref, pdst_ref):
  psrc_ref[...] = jnp.dot(mesh_ref[...], w1b_ref[...],
                          preferred_element_type=F32)
  pdst_ref[...] = jnp.dot(grid_ref[...], w1c_ref[...],
                          preferred_element_type=F32) + be1_ref[...]


def _project(mesh_nodes, grid_nodes, w1b, w1c, be1):
  blk = 2000
  return pl.pallas_call(
      _proj_body,
      grid=(NM // blk,),
      in_specs=[
          pl.BlockSpec((blk, H), lambda i: (i, 0)),
          pl.BlockSpec((blk, H), lambda i: (i, 0)),
          pl.BlockSpec((H, H), lambda i: (0, 0)),
          pl.BlockSpec((H, H), lambda i: (0, 0)),
          pl.BlockSpec((1, H), lambda i: (0, 0)),
      ],
      out_specs=[
          pl.BlockSpec((blk, H), lambda i: (i, 0)),
          pl.BlockSpec((blk, H), lambda i: (i, 0)),
      ],
      out_shape=[jax.ShapeDtypeStruct((NM, H), F32),
                 jax.ShapeDtypeStruct((NG, H), F32)],
  )(mesh_nodes, grid_nodes, w1b, w1c, be1)


def _sc_gather_body(psrc_hbm, pdst_hbm, srcr_hbm, dstr_hbm,
                    gs_hbm,
                    idxs_v, idxd_v, rows_v,
                    sg0, sg1, sw0, sw1):
  c = lax.axis_index("c")
  s = lax.axis_index("s")
  wid = c * NS + s
  ebase = wid * EPW
  pltpu.sync_copy(srcr_hbm.at[wid], idxs_v)
  pltpu.sync_copy(dstr_hbm.at[wid], idxd_v)
  sg = (sg0, sg1)
  sw = (sw0, sw1)

  def fire_src(m, p):
    for k in range(KG):
      pltpu.async_copy(psrc_hbm.at[idxs_v.at[m * KG + k]],
                       rows_v.at[p, pl.ds(k * CHG, CHG)], sg[p])

  def fire_add(m, p):
    for k in range(KG):
      pltpu.async_copy(pdst_hbm.at[idxd_v.at[m * KG + k]],
                       rows_v.at[p, pl.ds(k * CHG, CHG)], sg[p], add=True)

  def drain_g(p):
    for k in range(KG):
      pltpu.make_async_copy(psrc_hbm.at[idxs_v.at[k]],
                            rows_v.at[p, pl.ds(k * CHG, CHG)], sg[p]).wait()

  def wait_w(p):
    pltpu.make_async_copy(rows_v.at[p],
                          gs_hbm.at[pl.ds(ebase, SRG)], sw[p]).wait()

  def mid(m, p, q):
    drain_g(p)
    fire_add(m, p)
    @pl.when(m >= 1)
    def _():
      wait_w(q)

    @pl.when(m + 1 < NSUPG)
    def _():
      fire_src(m + 1, q)
    drain_g(p)
    pltpu.async_copy(rows_v.at[p],
                     gs_hbm.at[pl.ds(ebase + m * SRG, SRG)], sw[p])

  fire_src(0, 0)

  def step(mm, carry):
    for p in range(2):
      mid(2 * mm + p, p, 1 - p)
    return carry
  lax.fori_loop(0, NSUPG // 2, step, 0)
  mid(jnp.int32(NSUPG - 1), 0, 1)
  wait_w(0)


def _sc_gather(psrc, pdst, src_r, dst_r):
  fn = pl.kernel(
      _sc_gather_body,
      out_type=jax.ShapeDtypeStruct((NEH, H), F32),
      mesh=_mesh(),
      scratch_types=[
          pltpu.VMEM((NCHUNKG, CHG), jnp.int32),
          pltpu.VMEM((NCHUNKG, CHG), jnp.int32),
          pltpu.VMEM((2, SRG, H), F32),
          pltpu.SemaphoreType.DMA,
          pltpu.SemaphoreType.DMA,
          pltpu.SemaphoreType.DMA,
          pltpu.SemaphoreType.DMA,
      ],
  )
  return fn(psrc, pdst, src_r, dst_r)


def _edge_body(e_ref, gs_ref, w1a_ref, w2_ref, be2_ref,
               gam_ref, bet_ref, out_ref):
  e = e_ref[...]
  h = jnp.dot(e.astype(BF16), w1a_ref[...], preferred_element_type=F32)
  h = h + gs_ref[...]
  h = h * jax.nn.sigmoid(h)
  h = jnp.dot(h.astype(BF16), w2_ref[...],
              preferred_element_type=F32) + be2_ref[...]
  mu = jnp.mean(h, axis=-1, keepdims=True)
  var = jnp.mean((h - mu) * (h - mu), axis=-1, keepdims=True)
  out_ref[...] = ((h - mu) * lax.rsqrt(var + 1e-5) * gam_ref[...]
                  + bet_ref[...] + e)


def _edge_mlp(e_feats, half, gs, w1a, w2, be2, gam, bet):
  base = half * (NEH // EBLK)
  return pl.pallas_call(
      _edge_body,
      grid=(NEH // EBLK,),
      in_specs=[
          pl.BlockSpec((EBLK, H), lambda i: (i + base, 0)),
          pl.BlockSpec((EBLK, H), lambda i: (i, 0)),
          pl.BlockSpec((H, H), lambda i: (0, 0)),
          pl.BlockSpec((H, H), lambda i: (0, 0)),
          pl.BlockSpec((1, H), lambda i: (0, 0)),
          pl.BlockSpec((1, H), lambda i: (0, 0)),
          pl.BlockSpec((1, H), lambda i: (0, 0)),
      ],
      out_specs=pl.BlockSpec((EBLK, H), lambda i: (i, 0)),
      out_shape=jax.ShapeDtypeStruct((NEH, H), F32),
      compiler_params=pltpu.CompilerParams(
          dimension_semantics=("arbitrary",)),
  )(e_feats, gs, w1a, w2, be2, gam, bet)


def _sc_scatter_body(ef_hbm, dstw_hbm, out_hbm,
                     acc_sh, val_v, idx_v,
                     sv0, sv1, sa0, sa1):
  c = lax.axis_index("c")
  s = lax.axis_index("s")
  wid = c * NS + s
  ebase = wid * EPW
  sv = (sv0, sv1)
  sa = (sa0, sa1)

  zero = jnp.zeros((16,), F32)

  def zrow(r, carry):
    for k in range(H // 16):
      val_v[0, r, pl.ds(16 * k, 16)] = zero
    return carry
  lax.fori_loop(0, CHS, zrow, 0)
  rbase = s * ACC_PW
  for z in range(ACC_PW // CHS):
    pltpu.sync_copy(val_v.at[0],
                    acc_sh.at[pl.ds(rbase + z * CHS, CHS)])
  plsc.subcore_barrier()

  def fire_read(m, p):
    pltpu.async_copy(dstw_hbm.at[wid, m], idx_v.at[p], sv[p])
    pltpu.async_copy(ef_hbm.at[pl.ds(ebase + m * CHS, CHS)],
                     val_v.at[p], sv[p])

  def wait_read(p):
    pltpu.make_async_copy(dstw_hbm.at[wid, 0], idx_v.at[p], sv[p]).wait()
    pltpu.make_async_copy(ef_hbm.at[pl.ds(ebase, CHS)],
                          val_v.at[p], sv[p]).wait()

  def fire_adds(p):
    pltpu.async_copy(val_v.at[p], acc_sh.at[idx_v.at[p]], sa[p], add=True)

  def wait_adds(p):
    pltpu.make_async_copy(val_v.at[p], acc_sh.at[idx_v.at[p]],
                          sa[p]).wait()

  def smid(m, p, q):
    wait_read(p)
    @pl.when(m >= 1)
    def _():
      wait_adds(q)

    @pl.when(m + 1 < NSUPS)
    def _():
      fire_read(m + 1, q)
    fire_adds(p)

  fire_read(0, 0)

  def step(mm, carry):
    for p in range(2):
      smid(2 * mm + p, p, 1 - p)
    return carry
  lax.fori_loop(0, NSUPS // 2, step, 0)
  smid(jnp.int32(NSUPS - 1), 0, 1)
  wait_adds(0)
  plsc.subcore_barrier()

  for z in range(ACC_PW // CHS):
    pltpu.sync_copy(acc_sh.at[pl.ds(rbase + z * CHS, CHS)],
                    val_v.at[0])
    pltpu.sync_copy(val_v.at[0],
                    out_hbm.at[c, pl.ds(rbase + z * CHS, CHS)])


def _sc_scatter(ef, dst_w):
  fn = pl.kernel(
      _sc_scatter_body,
      out_type=jax.ShapeDtypeStruct((NC, NGP, H), F32),
      mesh=_mesh(),
      scratch_types=[
          pltpu.VMEM_SHARED((NGP, H), F32),
          pltpu.VMEM((2, CHS, H), F32),
          pltpu.VMEM((2, CHS), jnp.int32),
          pltpu.SemaphoreType.DMA,
          pltpu.SemaphoreType.DMA,
          pltpu.SemaphoreType.DMA,
          pltpu.SemaphoreType.DMA,
      ],
  )
  return fn(ef, dst_w)


def _node_body(g_ref, p0_ref, p1_ref, p2_ref, p3_ref,
               wn1a_ref, wn1b_ref, bn1_ref,
               wn2_ref, bn2_ref, gam_ref, bet_ref, out_ref):
  g = g_ref[...]
  agg = (p0_ref[0] + p1_ref[0]) + (p2_ref[0] + p3_ref[0])
  h = (jnp.dot(g.astype(BF16), wn1a_ref[...], preferred_element_type=F32)
       + jnp.dot(agg.astype(BF16), wn1b_ref[...],
                 preferred_element_type=F32)
       + bn1_ref[...])
  h = h * jax.nn.sigmoid(h)
  h = jnp.dot(h.astype(BF16), wn2_ref[...],
              preferred_element_type=F32) + bn2_ref[...]
  mu = jnp.mean(h, axis=-1, keepdims=True)
  var = jnp.mean((h - mu) * (h - mu), axis=-1, keepdims=True)
  out_ref[...] = ((h - mu) * lax.rsqrt(var + 1e-5) * gam_ref[...]
                  + bet_ref[...] + g)


def _node_mlp(grid_nodes, pa, pb, wn1a, wn1b, bn1, wn2, bn2, gam, bet):
  blk = 2000
  return pl.pallas_call(
      _node_body,
      grid=(NG // blk,),
      in_specs=[
          pl.BlockSpec((blk, H), lambda i: (i, 0)),
          pl.BlockSpec((1, blk, H), lambda i: (0, i, 0)),
          pl.BlockSpec((1, blk, H), lambda i: (1, i, 0)),
          pl.BlockSpec((1, blk, H), lambda i: (0, i, 0)),
          pl.BlockSpec((1, blk, H), lambda i: (1, i, 0)),
          pl.BlockSpec((H, H), lambda i: (0, 0)),
          pl.BlockSpec((H, H), lambda i: (0, 0)),
          pl.BlockSpec((1, H), lambda i: (0, 0)),
          pl.BlockSpec((H, H), lambda i: (0, 0)),
          pl.BlockSpec((1, H), lambda i: (0, 0)),
          pl.BlockSpec((1, H), lambda i: (0, 0)),
          pl.BlockSpec((1, H), lambda i: (0, 0)),
      ],
      out_specs=pl.BlockSpec((blk, H), lambda i: (i, 0)),
      out_shape=jax.ShapeDtypeStruct((NG, H), F32),
  )(grid_nodes, pa, pa, pb, pb, wn1a, wn1b, bn1, wn2, bn2, gam, bet)


def kernel(mesh2grid_edge_features, grid_node_features, mesh_node_features,
           mesh2grid_edge_indices_src, mesh2grid_edge_indices_dst,
           We1, be1, We2, be2, e_gamma, e_beta,
           Wn1, bn1, Wn2, bn2, n_gamma, n_beta):
  w1a = We1[:H]
  w1b = We1[H:2 * H]
  w1c = We1[2 * H:]
  wn1a = Wn1[:H]
  wn1b = Wn1[H:]
  r1 = lambda v: v.reshape(1, H)

  src = mesh2grid_edge_indices_src
  dst = mesh2grid_edge_indices_dst
  src_r = [src[:NEH].reshape(NW, NCHUNKG, CHG),
           src[NEH:].reshape(NW, NCHUNKG, CHG)]
  dst_r = [dst[:NEH].reshape(NW, NCHUNKG, CHG),
           dst[NEH:].reshape(NW, NCHUNKG, CHG)]
  dst_w = [dst[:NEH].reshape(NW, NSUPS, CHS),
           dst[NEH:].reshape(NW, NSUPS, CHS)]

  psrc, pdst = _project(mesh_node_features, grid_node_features,
                        w1b, w1c, r1(be1))

  w1a_b = w1a.astype(BF16)
  w2_b = We2.astype(BF16)
  e_args = (w1a_b, w2_b, r1(be2), r1(e_gamma), r1(e_beta))

  g0 = _sc_gather(psrc, pdst, src_r[0], dst_r[0])
  ef0 = _edge_mlp(mesh2grid_edge_features, 0, g0, *e_args)
  g1 = _sc_gather(psrc, pdst, src_r[1], dst_r[1])
  ef1 = _edge_mlp(mesh2grid_edge_features, 1, g1, *e_args)
  pa = _sc_scatter(ef0, dst_w[0])
  pb = _sc_scatter(ef1, dst_w[1])

  out = _node_mlp(grid_node_features, pa, pb,
                  wn1a.astype(BF16), wn1b.astype(BF16), r1(bn1),
                  Wn2.astype(BF16), r1(bn2), r1(n_gamma), r1(n_beta))
  return out


# speedup vs baseline: 1.0517x; 1.0517x over previous
import functools

import jax
import jax.numpy as jnp
from jax import lax
from jax.experimental import pallas as pl
from jax.experimental.pallas import tpu as pltpu
from jax.experimental.pallas import tpu_sc as plsc

F32 = jnp.float32
BF16 = jnp.bfloat16
NG = 10000
NM = 10000
NE = 320000
NEH = NE // 2
H = 128

NC = 2
NS = 16
NW = NC * NS

EPW = NEH // NW
CHG = 100
KG = 2
SRG = KG * CHG
NCHUNKG = EPW // CHG
NSUPG = EPW // SRG
CHS = 40
KS = 5
SRS = KS * CHS
NSUPS = EPW // SRS
NGP = 10240
ACC_PW = NGP // NS
ZCH = 64
EBLK = 1280


def _mesh():
  return plsc.VectorSubcoreMesh(core_axis_name="c", subcore_axis_name="s",
                                num_cores=NC, num_subcores=NS)


def _proj_body(mesh_ref, grid_ref, w1b_ref, w1c_ref, be1_ref,
               psrc_ref, pdst_ref):
  psrc_ref[...] = jnp.dot(mesh_ref[...], w1b_ref[...],
                          preferred_element_type=F32)
  pdst_ref[...] = jnp.dot(grid_ref[...], w1c_ref[...],
                          preferred_element_type=F32) + be1_ref[...]


def _project(mesh_nodes, grid_nodes, w1b, w1c, be1):
  blk = 2000
  return pl.pallas_call(
      _proj_body,
      grid=(NM // blk,),
      in_specs=[
          pl.BlockSpec((blk, H), lambda i: (i, 0)),
          pl.BlockSpec((blk, H), lambda i: (i, 0)),
          pl.BlockSpec((H, H), lambda i: (0, 0)),
          pl.BlockSpec((H, H), lambda i: (0, 0)),
          pl.BlockSpec((1, H), lambda i: (0, 0)),
      ],
      out_specs=[
          pl.BlockSpec((blk, H), lambda i: (i, 0)),
          pl.BlockSpec((blk, H), lambda i: (i, 0)),
      ],
      out_shape=[jax.ShapeDtypeStruct((NM, H), F32),
                 jax.ShapeDtypeStruct((NG, H), F32)],
  )(mesh_nodes, grid_nodes, w1b, w1c, be1)


def _sc_gather_body(psrc_hbm, pdst_hbm, srcr_hbm, dstr_hbm,
                    gs_hbm,
                    idxs_v, idxd_v, rows_v,
                    sg0, sg1, sw0, sw1):
  c = lax.axis_index("c")
  s = lax.axis_index("s")
  wid = c * NS + s
  ebase = wid * EPW
  pltpu.sync_copy(srcr_hbm.at[wid], idxs_v)
  pltpu.sync_copy(dstr_hbm.at[wid], idxd_v)
  sg = (sg0, sg1)
  sw = (sw0, sw1)

  def fire_src(m, p):
    for k in range(KG):
      pltpu.async_copy(psrc_hbm.at[idxs_v.at[m * KG + k]],
                       rows_v.at[p, pl.ds(k * CHG, CHG)], sg[p])

  def fire_add(m, p):
    for k in range(KG):
      pltpu.async_copy(pdst_hbm.at[idxd_v.at[m * KG + k]],
                       rows_v.at[p, pl.ds(k * CHG, CHG)], sg[p], add=True)

  def drain_g(p):
    for k in range(KG):
      pltpu.make_async_copy(psrc_hbm.at[idxs_v.at[k]],
                            rows_v.at[p, pl.ds(k * CHG, CHG)], sg[p]).wait()

  def wait_w(p):
    pltpu.make_async_copy(rows_v.at[p],
                          gs_hbm.at[pl.ds(ebase, SRG)], sw[p]).wait()

  def mid(m, p, q):
    drain_g(p)
    fire_add(m, p)
    @pl.when(m >= 1)
    def _():
      wait_w(q)

    @pl.when(m + 1 < NSUPG)
    def _():
      fire_src(m + 1, q)
    drain_g(p)
    pltpu.async_copy(rows_v.at[p],
                     gs_hbm.at[pl.ds(ebase + m * SRG, SRG)], sw[p])

  fire_src(0, 0)

  def step(mm, carry):
    for p in range(2):
      mid(2 * mm + p, p, 1 - p)
    return carry
  lax.fori_loop(0, NSUPG // 2, step, 0)
  mid(jnp.int32(NSUPG - 1), 0, 1)
  wait_w(0)


def _sc_gather(psrc, pdst, src_r, dst_r):
  fn = pl.kernel(
      _sc_gather_body,
      out_type=jax.ShapeDtypeStruct((NEH, H), F32),
      mesh=_mesh(),
      scratch_types=[
          pltpu.VMEM((NCHUNKG, CHG), jnp.int32),
          pltpu.VMEM((NCHUNKG, CHG), jnp.int32),
          pltpu.VMEM((2, SRG, H), F32),
          pltpu.SemaphoreType.DMA,
          pltpu.SemaphoreType.DMA,
          pltpu.SemaphoreType.DMA,
          pltpu.SemaphoreType.DMA,
      ],
  )
  return fn(psrc, pdst, src_r, dst_r)


def _edge_body(e_ref, gs_ref, w1a_ref, w2_ref, be2_ref,
               gam_ref, bet_ref, out_ref):
  e = e_ref[...]
  h = jnp.dot(e.astype(BF16), w1a_ref[...], preferred_element_type=F32)
  h = h + gs_ref[...]
  h = h * jax.nn.sigmoid(h)
  h = jnp.dot(h.astype(BF16), w2_ref[...],
              preferred_element_type=F32) + be2_ref[...]
  mu = jnp.mean(h, axis=-1, keepdims=True)
  var = jnp.mean((h - mu) * (h - mu), axis=-1, keepdims=True)
  out_ref[...] = ((h - mu) * lax.rsqrt(var + 1e-5) * gam_ref[...]
                  + bet_ref[...] + e)


def _edge_mlp(e_feats, half, gs, w1a, w2, be2, gam, bet):
  base = half * (NEH // EBLK)
  return pl.pallas_call(
      _edge_body,
      grid=(NEH // EBLK,),
      in_specs=[
          pl.BlockSpec((EBLK, H), lambda i: (i + base, 0)),
          pl.BlockSpec((EBLK, H), lambda i: (i, 0)),
          pl.BlockSpec((H, H), lambda i: (0, 0)),
          pl.BlockSpec((H, H), lambda i: (0, 0)),
          pl.BlockSpec((1, H), lambda i: (0, 0)),
          pl.BlockSpec((1, H), lambda i: (0, 0)),
          pl.BlockSpec((1, H), lambda i: (0, 0)),
      ],
      out_specs=pl.BlockSpec((EBLK, H), lambda i: (i, 0)),
      out_shape=jax.ShapeDtypeStruct((NEH, H), F32),
      compiler_params=pltpu.CompilerParams(
          dimension_semantics=("arbitrary",)),
  )(e_feats, gs, w1a, w2, be2, gam, bet)


def _sc_scatter_body(ef_hbm, dstw_hbm, out_hbm,
                     acc_sh, val_v, idx_v, sem_in):
  c = lax.axis_index("c")
  s = lax.axis_index("s")
  wid = c * NS + s
  ebase = wid * EPW

  zero = jnp.zeros((16,), F32)

  def zrow(r, carry):
    for k in range(H // 16):
      val_v[r, pl.ds(16 * k, 16)] = zero
    return carry
  lax.fori_loop(0, CHS, zrow, 0)
  rbase = s * ACC_PW
  for z in range(ACC_PW // CHS):
    pltpu.sync_copy(val_v.at[pl.ds(0, CHS)],
                    acc_sh.at[pl.ds(rbase + z * CHS, CHS)])
  plsc.subcore_barrier()

  def superchunk(m, carry):
    icp = pltpu.async_copy(dstw_hbm.at[wid, m], idx_v, sem_in)
    vcp = pltpu.async_copy(ef_hbm.at[pl.ds(ebase + m * SRS, SRS)],
                           val_v, sem_in)
    icp.wait()
    vcp.wait()
    for k in range(KS):
      pltpu.sync_copy(val_v.at[pl.ds(k * CHS, CHS)],
                      acc_sh.at[idx_v.at[k]], add=True)
    return carry
  lax.fori_loop(0, NSUPS, superchunk, 0)
  plsc.subcore_barrier()

  for z in range(ACC_PW // CHS):
    pltpu.sync_copy(acc_sh.at[pl.ds(rbase + z * CHS, CHS)],
                    val_v.at[pl.ds(0, CHS)])
    pltpu.sync_copy(val_v.at[pl.ds(0, CHS)],
                    out_hbm.at[c, pl.ds(rbase + z * CHS, CHS)])


def _sc_scatter(ef, dst_w):
  fn = pl.kernel(
      _sc_scatter_body,
      out_type=jax.ShapeDtypeStruct((NC, NGP, H), F32),
      mesh=_mesh(),
      scratch_types=[
          pltpu.VMEM_SHARED((NGP, H), F32),
          pltpu.VMEM((SRS, H), F32),
          pltpu.VMEM((KS, CHS), jnp.int32),
          pltpu.SemaphoreType.DMA,
      ],
  )
  return fn(ef, dst_w)


def _node_body(g_ref, p0_ref, p1_ref, p2_ref, p3_ref,
               wn1a_ref, wn1b_ref, bn1_ref,
               wn2_ref, bn2_ref, gam_ref, bet_ref, out_ref):
  g = g_ref[...]
  agg = (p0_ref[0] + p1_ref[0]) + (p2_ref[0] + p3_ref[0])
  h = (jnp.dot(g.astype(BF16), wn1a_ref[...], preferred_element_type=F32)
       + jnp.dot(agg.astype(BF16), wn1b_ref[...],
                 preferred_element_type=F32)
       + bn1_ref[...])
  h = h * jax.nn.sigmoid(h)
  h = jnp.dot(h.astype(BF16), wn2_ref[...],
              preferred_element_type=F32) + bn2_ref[...]
  mu = jnp.mean(h, axis=-1, keepdims=True)
  var = jnp.mean((h - mu) * (h - mu), axis=-1, keepdims=True)
  out_ref[...] = ((h - mu) * lax.rsqrt(var + 1e-5) * gam_ref[...]
                  + bet_ref[...] + g)


def _node_mlp(grid_nodes, pa, pb, wn1a, wn1b, bn1, wn2, bn2, gam, bet):
  blk = 2000
  return pl.pallas_call(
      _node_body,
      grid=(NG // blk,),
      in_specs=[
          pl.BlockSpec((blk, H), lambda i: (i, 0)),
          pl.BlockSpec((1, blk, H), lambda i: (0, i, 0)),
          pl.BlockSpec((1, blk, H), lambda i: (1, i, 0)),
          pl.BlockSpec((1, blk, H), lambda i: (0, i, 0)),
          pl.BlockSpec((1, blk, H), lambda i: (1, i, 0)),
          pl.BlockSpec((H, H), lambda i: (0, 0)),
          pl.BlockSpec((H, H), lambda i: (0, 0)),
          pl.BlockSpec((1, H), lambda i: (0, 0)),
          pl.BlockSpec((H, H), lambda i: (0, 0)),
          pl.BlockSpec((1, H), lambda i: (0, 0)),
          pl.BlockSpec((1, H), lambda i: (0, 0)),
          pl.BlockSpec((1, H), lambda i: (0, 0)),
      ],
      out_specs=pl.BlockSpec((blk, H), lambda i: (i, 0)),
      out_shape=jax.ShapeDtypeStruct((NG, H), F32),
  )(grid_nodes, pa, pa, pb, pb, wn1a, wn1b, bn1, wn2, bn2, gam, bet)


def kernel(mesh2grid_edge_features, grid_node_features, mesh_node_features,
           mesh2grid_edge_indices_src, mesh2grid_edge_indices_dst,
           We1, be1, We2, be2, e_gamma, e_beta,
           Wn1, bn1, Wn2, bn2, n_gamma, n_beta):
  w1a = We1[:H]
  w1b = We1[H:2 * H]
  w1c = We1[2 * H:]
  wn1a = Wn1[:H]
  wn1b = Wn1[H:]
  r1 = lambda v: v.reshape(1, H)

  src = mesh2grid_edge_indices_src
  dst = mesh2grid_edge_indices_dst
  src_r = [src[:NEH].reshape(NW, NCHUNKG, CHG),
           src[NEH:].reshape(NW, NCHUNKG, CHG)]
  dst_r = [dst[:NEH].reshape(NW, NCHUNKG, CHG),
           dst[NEH:].reshape(NW, NCHUNKG, CHG)]
  dst_w = [dst[:NEH].reshape(NW, NSUPS, KS, CHS),
           dst[NEH:].reshape(NW, NSUPS, KS, CHS)]

  psrc, pdst = _project(mesh_node_features, grid_node_features,
                        w1b, w1c, r1(be1))

  w1a_b = w1a.astype(BF16)
  w2_b = We2.astype(BF16)
  e_args = (w1a_b, w2_b, r1(be2), r1(e_gamma), r1(e_beta))

  g0 = _sc_gather(psrc, pdst, src_r[0], dst_r[0])
  ef0 = _edge_mlp(mesh2grid_edge_features, 0, g0, *e_args)
  g1 = _sc_gather(psrc, pdst, src_r[1], dst_r[1])
  ef1 = _edge_mlp(mesh2grid_edge_features, 1, g1, *e_args)
  pa = _sc_scatter(ef0, dst_w[0])
  pb = _sc_scatter(ef1, dst_w[1])

  out = _node_mlp(grid_node_features, pa, pb,
                  wn1a.astype(BF16), wn1b.astype(BF16), r1(bn1),
                  Wn2.astype(BF16), r1(bn2), r1(n_gamma), r1(n_beta))
  return out


# speedup vs baseline: 1.0621x; 1.0098x over previous
import functools

import jax
import jax.numpy as jnp
from jax import lax
from jax.experimental import pallas as pl
from jax.experimental.pallas import tpu as pltpu
from jax.experimental.pallas import tpu_sc as plsc

F32 = jnp.float32
BF16 = jnp.bfloat16
NG = 10000
NM = 10000
NE = 320000
NEH = NE // 2
H = 128

NC = 2
NS = 16
NW = NC * NS

EPW = NEH // NW
CHG = 100
KG = 2
SRG = KG * CHG
NCHUNKG = EPW // CHG
NSUPG = EPW // SRG
CHS = 40
KS = 5
SRS = KS * CHS
NSUPS = EPW // SRS
NGP = 10240
ACC_PW = NGP // NS
ZCH = 64
EBLK = 1280


def _mesh():
  return plsc.VectorSubcoreMesh(core_axis_name="c", subcore_axis_name="s",
                                num_cores=NC, num_subcores=NS)


def _proj_body(mesh_ref, grid_ref, w1b_ref, w1c_ref, be1_ref,
               psrc_ref, pdst_ref):
  psrc_ref[...] = jnp.dot(mesh_ref[...], w1b_ref[...],
                          preferred_element_type=F32)
  pdst_ref[...] = jnp.dot(grid_ref[...], w1c_ref[...],
                          preferred_element_type=F32) + be1_ref[...]


def _project(mesh_nodes, grid_nodes, w1b, w1c, be1):
  blk = 2000
  return pl.pallas_call(
      _proj_body,
      grid=(NM // blk,),
      in_specs=[
          pl.BlockSpec((blk, H), lambda i: (i, 0)),
          pl.BlockSpec((blk, H), lambda i: (i, 0)),
          pl.BlockSpec((H, H), lambda i: (0, 0)),
          pl.BlockSpec((H, H), lambda i: (0, 0)),
          pl.BlockSpec((1, H), lambda i: (0, 0)),
      ],
      out_specs=[
          pl.BlockSpec((blk, H), lambda i: (i, 0)),
          pl.BlockSpec((blk, H), lambda i: (i, 0)),
      ],
      out_shape=[jax.ShapeDtypeStruct((NM, H), F32),
                 jax.ShapeDtypeStruct((NG, H), F32)],
  )(mesh_nodes, grid_nodes, w1b, w1c, be1)


def _sc_gather_body(psrc_hbm, pdst_hbm, srcr_hbm, dstr_hbm,
                    gs_hbm,
                    idxs_v, idxd_v, rows_v,
                    sg0, sg1, sw0, sw1):
  c = lax.axis_index("c")
  s = lax.axis_index("s")
  wid = c * NS + s
  ebase = wid * EPW
  pltpu.sync_copy(srcr_hbm.at[wid], idxs_v)
  pltpu.sync_copy(dstr_hbm.at[wid], idxd_v)
  sg = (sg0, sg1)
  sw = (sw0, sw1)

  def fire_src(m, p):
    for k in range(KG):
      pltpu.async_copy(psrc_hbm.at[idxs_v.at[m * KG + k]],
                       rows_v.at[p, pl.ds(k * CHG, CHG)], sg[p])

  def fire_add(m, p):
    for k in range(KG):
      pltpu.async_copy(pdst_hbm.at[idxd_v.at[m * KG + k]],
                       rows_v.at[p, pl.ds(k * CHG, CHG)], sg[p], add=True)

  def drain_g(p):
    for k in range(KG):
      pltpu.make_async_copy(psrc_hbm.at[idxs_v.at[k]],
                            rows_v.at[p, pl.ds(k * CHG, CHG)], sg[p]).wait()

  def wait_w(p):
    pltpu.make_async_copy(rows_v.at[p],
                          gs_hbm.at[pl.ds(ebase, SRG)], sw[p]).wait()

  def mid(m, p, q):
    drain_g(p)
    fire_add(m, p)
    @pl.when(m >= 1)
    def _():
      wait_w(q)

    @pl.when(m + 1 < NSUPG)
    def _():
      fire_src(m + 1, q)
    drain_g(p)
    pltpu.async_copy(rows_v.at[p],
                     gs_hbm.at[pl.ds(ebase + m * SRG, SRG)], sw[p])

  fire_src(0, 0)

  def step(mm, carry):
    for p in range(2):
      mid(2 * mm + p, p, 1 - p)
    return carry
  lax.fori_loop(0, NSUPG // 2, step, 0)
  mid(jnp.int32(NSUPG - 1), 0, 1)
  wait_w(0)


def _sc_gather(psrc, pdst, src_r, dst_r):
  fn = pl.kernel(
      _sc_gather_body,
      out_type=jax.ShapeDtypeStruct((NEH, H), F32),
      mesh=_mesh(),
      scratch_types=[
          pltpu.VMEM((NCHUNKG, CHG), jnp.int32),
          pltpu.VMEM((NCHUNKG, CHG), jnp.int32),
          pltpu.VMEM((2, SRG, H), F32),
          pltpu.SemaphoreType.DMA,
          pltpu.SemaphoreType.DMA,
          pltpu.SemaphoreType.DMA,
          pltpu.SemaphoreType.DMA,
      ],
  )
  return fn(psrc, pdst, src_r, dst_r)


def _edge_body(e_ref, gs_ref, w1a_ref, w2_ref, be2_ref,
               gam_ref, bet_ref, out_ref):
  e = e_ref[...]
  h = jnp.dot(e.astype(BF16), w1a_ref[...], preferred_element_type=F32)
  h = h + gs_ref[...]
  h = h * jax.nn.sigmoid(h)
  h = jnp.dot(h.astype(BF16), w2_ref[...],
              preferred_element_type=F32) + be2_ref[...]
  mu = jnp.mean(h, axis=-1, keepdims=True)
  ms = jnp.mean(h * h, axis=-1, keepdims=True)
  out_ref[...] = ((h - mu) * lax.rsqrt(ms - mu * mu + 1e-5) * gam_ref[...]
                  + bet_ref[...] + e)


def _edge_mlp(e_feats, half, gs, w1a, w2, be2, gam, bet):
  base = half * (NEH // EBLK)
  return pl.pallas_call(
      _edge_body,
      grid=(NEH // EBLK,),
      in_specs=[
          pl.BlockSpec((EBLK, H), lambda i: (i + base, 0)),
          pl.BlockSpec((EBLK, H), lambda i: (i, 0)),
          pl.BlockSpec((H, H), lambda i: (0, 0)),
          pl.BlockSpec((H, H), lambda i: (0, 0)),
          pl.BlockSpec((1, H), lambda i: (0, 0)),
          pl.BlockSpec((1, H), lambda i: (0, 0)),
          pl.BlockSpec((1, H), lambda i: (0, 0)),
      ],
      out_specs=pl.BlockSpec((EBLK, H), lambda i: (i, 0)),
      out_shape=jax.ShapeDtypeStruct((NEH, H), F32),
      compiler_params=pltpu.CompilerParams(
          dimension_semantics=("arbitrary",)),
  )(e_feats, gs, w1a, w2, be2, gam, bet)


def _sc_scatter_body(ef_hbm, dstw_hbm, out_hbm,
                     acc_sh, val_v, idx_v, sem_in):
  c = lax.axis_index("c")
  s = lax.axis_index("s")
  wid = c * NS + s
  ebase = wid * EPW

  zero = jnp.zeros((16,), F32)

  def zrow(r, carry):
    for k in range(H // 16):
      val_v[r, pl.ds(16 * k, 16)] = zero
    return carry
  lax.fori_loop(0, CHS, zrow, 0)
  rbase = s * ACC_PW
  for z in range(ACC_PW // CHS):
    pltpu.sync_copy(val_v.at[pl.ds(0, CHS)],
                    acc_sh.at[pl.ds(rbase + z * CHS, CHS)])
  plsc.subcore_barrier()

  def superchunk(m, carry):
    icp = pltpu.async_copy(dstw_hbm.at[wid, m], idx_v, sem_in)
    vcp = pltpu.async_copy(ef_hbm.at[pl.ds(ebase + m * SRS, SRS)],
                           val_v, sem_in)
    icp.wait()
    vcp.wait()
    adds = []
    for k in range(KS):
      adds.append(pltpu.async_copy(val_v.at[pl.ds(k * CHS, CHS)],
                                   acc_sh.at[idx_v.at[k]], sem_in,
                                   add=True))
    for cp in adds:
      cp.wait()
    return carry
  lax.fori_loop(0, NSUPS, superchunk, 0)
  plsc.subcore_barrier()

  for z in range(ACC_PW // CHS):
    pltpu.sync_copy(acc_sh.at[pl.ds(rbase + z * CHS, CHS)],
                    val_v.at[pl.ds(0, CHS)])
    pltpu.sync_copy(val_v.at[pl.ds(0, CHS)],
                    out_hbm.at[c, pl.ds(rbase + z * CHS, CHS)])


def _sc_scatter(ef, dst_w):
  fn = pl.kernel(
      _sc_scatter_body,
      out_type=jax.ShapeDtypeStruct((NC, NGP, H), F32),
      mesh=_mesh(),
      scratch_types=[
          pltpu.VMEM_SHARED((NGP, H), F32),
          pltpu.VMEM((SRS, H), F32),
          pltpu.VMEM((KS, CHS), jnp.int32),
          pltpu.SemaphoreType.DMA,
      ],
  )
  return fn(ef, dst_w)


def _node_body(g_ref, p0_ref, p1_ref, p2_ref, p3_ref,
               wn1a_ref, wn1b_ref, bn1_ref,
               wn2_ref, bn2_ref, gam_ref, bet_ref, out_ref):
  g = g_ref[...]
  agg = (p0_ref[0] + p1_ref[0]) + (p2_ref[0] + p3_ref[0])
  h = (jnp.dot(g.astype(BF16), wn1a_ref[...], preferred_element_type=F32)
       + jnp.dot(agg.astype(BF16), wn1b_ref[...],
                 preferred_element_type=F32)
       + bn1_ref[...])
  h = h * jax.nn.sigmoid(h)
  h = jnp.dot(h.astype(BF16), wn2_ref[...],
              preferred_element_type=F32) + bn2_ref[...]
  mu = jnp.mean(h, axis=-1, keepdims=True)
  var = jnp.mean((h - mu) * (h - mu), axis=-1, keepdims=True)
  out_ref[...] = ((h - mu) * lax.rsqrt(var + 1e-5) * gam_ref[...]
                  + bet_ref[...] + g)


def _node_mlp(grid_nodes, pa, pb, wn1a, wn1b, bn1, wn2, bn2, gam, bet):
  blk = 2000
  return pl.pallas_call(
      _node_body,
      grid=(NG // blk,),
      in_specs=[
          pl.BlockSpec((blk, H), lambda i: (i, 0)),
          pl.BlockSpec((1, blk, H), lambda i: (0, i, 0)),
          pl.BlockSpec((1, blk, H), lambda i: (1, i, 0)),
          pl.BlockSpec((1, blk, H), lambda i: (0, i, 0)),
          pl.BlockSpec((1, blk, H), lambda i: (1, i, 0)),
          pl.BlockSpec((H, H), lambda i: (0, 0)),
          pl.BlockSpec((H, H), lambda i: (0, 0)),
          pl.BlockSpec((1, H), lambda i: (0, 0)),
          pl.BlockSpec((H, H), lambda i: (0, 0)),
          pl.BlockSpec((1, H), lambda i: (0, 0)),
          pl.BlockSpec((1, H), lambda i: (0, 0)),
          pl.BlockSpec((1, H), lambda i: (0, 0)),
      ],
      out_specs=pl.BlockSpec((blk, H), lambda i: (i, 0)),
      out_shape=jax.ShapeDtypeStruct((NG, H), F32),
  )(grid_nodes, pa, pa, pb, pb, wn1a, wn1b, bn1, wn2, bn2, gam, bet)


def kernel(mesh2grid_edge_features, grid_node_features, mesh_node_features,
           mesh2grid_edge_indices_src, mesh2grid_edge_indices_dst,
           We1, be1, We2, be2, e_gamma, e_beta,
           Wn1, bn1, Wn2, bn2, n_gamma, n_beta):
  w1a = We1[:H]
  w1b = We1[H:2 * H]
  w1c = We1[2 * H:]
  wn1a = Wn1[:H]
  wn1b = Wn1[H:]
  r1 = lambda v: v.reshape(1, H)

  src = mesh2grid_edge_indices_src
  dst = mesh2grid_edge_indices_dst
  src_r = [src[:NEH].reshape(NW, NCHUNKG, CHG),
           src[NEH:].reshape(NW, NCHUNKG, CHG)]
  dst_r = [dst[:NEH].reshape(NW, NCHUNKG, CHG),
           dst[NEH:].reshape(NW, NCHUNKG, CHG)]
  dst_w = [dst[:NEH].reshape(NW, NSUPS, KS, CHS),
           dst[NEH:].reshape(NW, NSUPS, KS, CHS)]

  psrc, pdst = _project(mesh_node_features, grid_node_features,
                        w1b, w1c, r1(be1))

  w1a_b = w1a.astype(BF16)
  w2_b = We2.astype(BF16)
  e_args = (w1a_b, w2_b, r1(be2), r1(e_gamma), r1(e_beta))

  g0 = _sc_gather(psrc, pdst, src_r[0], dst_r[0])
  ef0 = _edge_mlp(mesh2grid_edge_features, 0, g0, *e_args)
  g1 = _sc_gather(psrc, pdst, src_r[1], dst_r[1])
  ef1 = _edge_mlp(mesh2grid_edge_features, 1, g1, *e_args)
  pa = _sc_scatter(ef0, dst_w[0])
  pb = _sc_scatter(ef1, dst_w[1])

  out = _node_mlp(grid_node_features, pa, pb,
                  wn1a.astype(BF16), wn1b.astype(BF16), r1(bn1),
                  Wn2.astype(BF16), r1(bn2), r1(n_gamma), r1(n_beta))
  return out


# speedup vs baseline: 1.1395x; 1.0729x over previous
import functools

import jax
import jax.numpy as jnp
from jax import lax
from jax.experimental import pallas as pl
from jax.experimental.pallas import tpu as pltpu
from jax.experimental.pallas import tpu_sc as plsc

F32 = jnp.float32
BF16 = jnp.bfloat16
NG = 10000
NM = 10000
NE = 320000
NEH = NE // 2
H = 128

NC = 2
NS = 16
NW = NC * NS

NGP = 10240
ACC_PW = NGP // NS
ZC = 40
EBLK = 1280

GEO_A = dict(epw=2560, chg=80, kg=2, nsupg=16, chs=40, ks=4, nsups=16)
GEO_B = dict(epw=2320, chg=116, kg=2, nsupg=10, chs=116, ks=2, nsups=10)
SLICES = [(0, GEO_A), (81920, GEO_A), (163840, GEO_A), (245760, GEO_B)]


def _mesh():
  return plsc.VectorSubcoreMesh(core_axis_name="c", subcore_axis_name="s",
                                num_cores=NC, num_subcores=NS)


def _proj_body(mesh_ref, grid_ref, w1b_ref, w1c_ref, be1_ref,
               psrc_ref, pdst_ref):
  psrc_ref[...] = jnp.dot(mesh_ref[...], w1b_ref[...],
                          preferred_element_type=F32)
  pdst_ref[...] = jnp.dot(grid_ref[...], w1c_ref[...],
                          preferred_element_type=F32) + be1_ref[...]


def _project(mesh_nodes, grid_nodes, w1b, w1c, be1):
  blk = 2000
  return pl.pallas_call(
      _proj_body,
      grid=(NM // blk,),
      in_specs=[
          pl.BlockSpec((blk, H), lambda i: (i, 0)),
          pl.BlockSpec((blk, H), lambda i: (i, 0)),
          pl.BlockSpec((H, H), lambda i: (0, 0)),
          pl.BlockSpec((H, H), lambda i: (0, 0)),
          pl.BlockSpec((1, H), lambda i: (0, 0)),
      ],
      out_specs=[
          pl.BlockSpec((blk, H), lambda i: (i, 0)),
          pl.BlockSpec((blk, H), lambda i: (i, 0)),
      ],
      out_shape=[jax.ShapeDtypeStruct((NM, H), F32),
                 jax.ShapeDtypeStruct((NG, H), F32)],
  )(mesh_nodes, grid_nodes, w1b, w1c, be1)


def _make_gather(geo):
  epw, chg, kg, nsupg = geo["epw"], geo["chg"], geo["kg"], geo["nsupg"]
  srg = kg * chg
  nchunkg = epw // chg
  neh = epw * NW

  def body(psrc_hbm, pdst_hbm, srcr_hbm, dstr_hbm, gs_hbm,
           idxs_v, idxd_v, rows_v, sg0, sg1, sw0, sw1):
    c = lax.axis_index("c")
    s = lax.axis_index("s")
    wid = c * NS + s
    ebase = wid * epw
    pltpu.sync_copy(srcr_hbm.at[wid], idxs_v)
    pltpu.sync_copy(dstr_hbm.at[wid], idxd_v)
    sg = (sg0, sg1)
    sw = (sw0, sw1)

    def fire_src(m, p):
      for k in range(kg):
        pltpu.async_copy(psrc_hbm.at[idxs_v.at[m * kg + k]],
                         rows_v.at[p, pl.ds(k * chg, chg)], sg[p])

    def fire_add(m, p):
      for k in range(kg):
        pltpu.async_copy(pdst_hbm.at[idxd_v.at[m * kg + k]],
                         rows_v.at[p, pl.ds(k * chg, chg)], sg[p],
                         add=True)

    def drain_g(p):
      for k in range(kg):
        pltpu.make_async_copy(psrc_hbm.at[idxs_v.at[k]],
                              rows_v.at[p, pl.ds(k * chg, chg)],
                              sg[p]).wait()

    def wait_w(p):
      pltpu.make_async_copy(rows_v.at[p],
                            gs_hbm.at[pl.ds(ebase, srg)], sw[p]).wait()

    def mid(m, p, q):
      drain_g(p)
      fire_add(m, p)
      @pl.when(m >= 1)
      def _():
        wait_w(q)

      @pl.when(m + 1 < nsupg)
      def _():
        fire_src(m + 1, q)
      drain_g(p)
      pltpu.async_copy(rows_v.at[p],
                       gs_hbm.at[pl.ds(ebase + m * srg, srg)], sw[p])

    fire_src(0, 0)

    def step(mm, carry):
      for p in range(2):
        mid(2 * mm + p, p, 1 - p)
      return carry
    lax.fori_loop(0, nsupg // 2, step, 0)
    if nsupg % 2:
      mid(jnp.int32(nsupg - 1), 0, 1)
      wait_w(0)
    else:
      wait_w(1)

  return pl.kernel(
      body,
      out_type=jax.ShapeDtypeStruct((neh, H), F32),
      mesh=_mesh(),
      scratch_types=[
          pltpu.VMEM((nchunkg, chg), jnp.int32),
          pltpu.VMEM((nchunkg, chg), jnp.int32),
          pltpu.VMEM((2, srg, H), F32),
          pltpu.SemaphoreType.DMA,
          pltpu.SemaphoreType.DMA,
          pltpu.SemaphoreType.DMA,
          pltpu.SemaphoreType.DMA,
      ],
  )


def _edge_body(e_ref, gs_ref, w1a_ref, w2_ref, be2_ref,
               gam_ref, bet_ref, out_ref):
  e = e_ref[...]
  h = jnp.dot(e.astype(BF16), w1a_ref[...], preferred_element_type=F32)
  h = h + gs_ref[...]
  h = h * jax.nn.sigmoid(h)
  h = jnp.dot(h.astype(BF16), w2_ref[...],
              preferred_element_type=F32) + be2_ref[...]
  mu = jnp.mean(h, axis=-1, keepdims=True)
  ms = jnp.mean(h * h, axis=-1, keepdims=True)
  out_ref[...] = ((h - mu) * lax.rsqrt(ms - mu * mu + 1e-5) * gam_ref[...]
                  + bet_ref[...] + e)


def _edge_mlp(e_feats, off, neh, gs, w1a, w2, be2, gam, bet):
  base = off // EBLK
  return pl.pallas_call(
      _edge_body,
      grid=(neh // EBLK,),
      in_specs=[
          pl.BlockSpec((EBLK, H), lambda i: (i + base, 0)),
          pl.BlockSpec((EBLK, H), lambda i: (i, 0)),
          pl.BlockSpec((H, H), lambda i: (0, 0)),
          pl.BlockSpec((H, H), lambda i: (0, 0)),
          pl.BlockSpec((1, H), lambda i: (0, 0)),
          pl.BlockSpec((1, H), lambda i: (0, 0)),
          pl.BlockSpec((1, H), lambda i: (0, 0)),
      ],
      out_specs=pl.BlockSpec((EBLK, H), lambda i: (i, 0)),
      out_shape=jax.ShapeDtypeStruct((neh, H), F32),
      compiler_params=pltpu.CompilerParams(
          dimension_semantics=("arbitrary",)),
  )(e_feats, gs, w1a, w2, be2, gam, bet)


def _make_scatter(geo):
  epw, chs, ks, nsups = geo["epw"], geo["chs"], geo["ks"], geo["nsups"]
  srs = ks * chs

  def body(ef_hbm, dstw_hbm, out_hbm, acc_sh, val_v, idx_v, sem_in):
    c = lax.axis_index("c")
    s = lax.axis_index("s")
    wid = c * NS + s
    ebase = wid * epw

    zero = jnp.zeros((16,), F32)

    def zrow(r, carry):
      for k in range(H // 16):
        val_v[r, pl.ds(16 * k, 16)] = zero
      return carry
    lax.fori_loop(0, ZC, zrow, 0)
    rbase = s * ACC_PW
    for z in range(ACC_PW // ZC):
      pltpu.sync_copy(val_v.at[pl.ds(0, ZC)],
                      acc_sh.at[pl.ds(rbase + z * ZC, ZC)])
    plsc.subcore_barrier()

    def superchunk(m, carry):
      icp = pltpu.async_copy(dstw_hbm.at[wid, m], idx_v, sem_in)
      vcp = pltpu.async_copy(ef_hbm.at[pl.ds(ebase + m * srs, srs)],
                             val_v, sem_in)
      icp.wait()
      vcp.wait()
      adds = []
      for k in range(ks):
        adds.append(pltpu.async_copy(val_v.at[pl.ds(k * chs, chs)],
                                     acc_sh.at[idx_v.at[k]], sem_in,
                                     add=True))
      for cp in adds:
        cp.wait()
      return carry
    lax.fori_loop(0, nsups, superchunk, 0)
    plsc.subcore_barrier()

    for z in range(ACC_PW // ZC):
      pltpu.sync_copy(acc_sh.at[pl.ds(rbase + z * ZC, ZC)],
                      val_v.at[pl.ds(0, ZC)])
      pltpu.sync_copy(val_v.at[pl.ds(0, ZC)],
                      out_hbm.at[c, pl.ds(rbase + z * ZC, ZC)])

  return pl.kernel(
      body,
      out_type=jax.ShapeDtypeStruct((NC, NGP, H), F32),
      mesh=_mesh(),
      scratch_types=[
          pltpu.VMEM_SHARED((NGP, H), F32),
          pltpu.VMEM((srs, H), F32),
          pltpu.VMEM((ks, chs), jnp.int32),
          pltpu.SemaphoreType.DMA,
      ],
  )


def _node_body(g_ref, p0_ref, p1_ref, p2_ref, p3_ref,
               wn1a_ref, wn1b_ref, bn1_ref,
               wn2_ref, bn2_ref, gam_ref, bet_ref, out_ref):
  g = g_ref[...]
  agg = ((p0_ref[0] + p0_ref[1]) + (p1_ref[0] + p1_ref[1])
         + (p2_ref[0] + p2_ref[1]) + (p3_ref[0] + p3_ref[1]))
  h = (jnp.dot(g.astype(BF16), wn1a_ref[...], preferred_element_type=F32)
       + jnp.dot(agg.astype(BF16), wn1b_ref[...],
                 preferred_element_type=F32)
       + bn1_ref[...])
  h = h * jax.nn.sigmoid(h)
  h = jnp.dot(h.astype(BF16), wn2_ref[...],
              preferred_element_type=F32) + bn2_ref[...]
  mu = jnp.mean(h, axis=-1, keepdims=True)
  var = jnp.mean((h - mu) * (h - mu), axis=-1, keepdims=True)
  out_ref[...] = ((h - mu) * lax.rsqrt(var + 1e-5) * gam_ref[...]
                  + bet_ref[...] + g)


def _node_mlp(grid_nodes, parts, wn1a, wn1b, bn1, wn2, bn2, gam, bet):
  blk = 2000
  return pl.pallas_call(
      _node_body,
      grid=(NG // blk,),
      in_specs=[
          pl.BlockSpec((blk, H), lambda i: (i, 0)),
          pl.BlockSpec((NC, blk, H), lambda i: (0, i, 0)),
          pl.BlockSpec((NC, blk, H), lambda i: (0, i, 0)),
          pl.BlockSpec((NC, blk, H), lambda i: (0, i, 0)),
          pl.BlockSpec((NC, blk, H), lambda i: (0, i, 0)),
          pl.BlockSpec((H, H), lambda i: (0, 0)),
          pl.BlockSpec((H, H), lambda i: (0, 0)),
          pl.BlockSpec((1, H), lambda i: (0, 0)),
          pl.BlockSpec((H, H), lambda i: (0, 0)),
          pl.BlockSpec((1, H), lambda i: (0, 0)),
          pl.BlockSpec((1, H), lambda i: (0, 0)),
          pl.BlockSpec((1, H), lambda i: (0, 0)),
      ],
      out_specs=pl.BlockSpec((blk, H), lambda i: (i, 0)),
      out_shape=jax.ShapeDtypeStruct((NG, H), F32),
  )(grid_nodes, *parts, wn1a, wn1b, bn1, wn2, bn2, gam, bet)


def kernel(mesh2grid_edge_features, grid_node_features, mesh_node_features,
           mesh2grid_edge_indices_src, mesh2grid_edge_indices_dst,
           We1, be1, We2, be2, e_gamma, e_beta,
           Wn1, bn1, Wn2, bn2, n_gamma, n_beta):
  w1a = We1[:H]
  w1b = We1[H:2 * H]
  w1c = We1[2 * H:]
  wn1a = Wn1[:H]
  wn1b = Wn1[H:]
  r1 = lambda v: v.reshape(1, H)

  src = mesh2grid_edge_indices_src
  dst = mesh2grid_edge_indices_dst

  psrc, pdst = _project(mesh_node_features, grid_node_features,
                        w1b, w1c, r1(be1))

  w1a_b = w1a.astype(BF16)
  w2_b = We2.astype(BF16)
  e_args = (w1a_b, w2_b, r1(be2), r1(e_gamma), r1(e_beta))

  gather_fns = {id(GEO_A): _make_gather(GEO_A), id(GEO_B): _make_gather(GEO_B)}
  scatter_fns = {id(GEO_A): _make_scatter(GEO_A),
                 id(GEO_B): _make_scatter(GEO_B)}

  efs, dws, geos = [], [], []
  for off, geo in SLICES:
    epw, chg, kg = geo["epw"], geo["chg"], geo["kg"]
    chs, ks = geo["chs"], geo["ks"]
    neh = epw * NW
    end = off + neh
    nchunkg = epw // chg
    src_r = src[off:end].reshape(NW, nchunkg, chg)
    dst_r = dst[off:end].reshape(NW, nchunkg, chg)
    dws.append(dst[off:end].reshape(NW, epw // (ks * chs), ks, chs))
    geos.append(geo)
    g = gather_fns[id(geo)](psrc, pdst, src_r, dst_r)
    efs.append(_edge_mlp(mesh2grid_edge_features, off, neh, g, *e_args))

  parts = [scatter_fns[id(geo)](ef, dw)
           for ef, dw, geo in zip(efs, dws, geos)]

  out = _node_mlp(grid_node_features, parts,
                  wn1a.astype(BF16), wn1b.astype(BF16), r1(bn1),
                  Wn2.astype(BF16), r1(bn2), r1(n_gamma), r1(n_beta))
  return out


# speedup vs baseline: 1.1435x; 1.0034x over previous
import jax
import jax.numpy as jnp
from jax import lax
from jax.experimental import pallas as pl
from jax.experimental.pallas import tpu as pltpu
from jax.experimental.pallas import tpu_sc as plsc

F32 = jnp.float32
BF16 = jnp.bfloat16
NG = 10000
NM = 10000
NE = 320000
H = 128

NC = 2
NS = 16
NW = NC * NS

NGP = 10240
ACC_PW = NGP // NS
ZC = 40
EBLK = 1280

GEO_A = dict(epw=2560, chg=80, kg=2, nsupg=16, chs=40, ks=4, nsups=16)
GEO_B = dict(epw=2320, chg=116, kg=2, nsupg=10, chs=116, ks=2, nsups=10)
SLICES = [(0, GEO_A), (81920, GEO_A), (163840, GEO_A), (245760, GEO_B)]


def _mesh():
  return plsc.VectorSubcoreMesh(core_axis_name="c", subcore_axis_name="s",
                                num_cores=NC, num_subcores=NS)


def _proj_body(mesh_ref, grid_ref, w1b_ref, w1c_ref, be1_ref,
               psrc_ref, pdst_ref):
  psrc_ref[...] = jnp.dot(mesh_ref[...], w1b_ref[...],
                          preferred_element_type=F32)
  pdst_ref[...] = jnp.dot(grid_ref[...], w1c_ref[...],
                          preferred_element_type=F32) + be1_ref[...]


def _project(mesh_nodes, grid_nodes, w1b, w1c, be1):
  blk = 2000
  return pl.pallas_call(
      _proj_body,
      grid=(NM // blk,),
      in_specs=[
          pl.BlockSpec((blk, H), lambda i: (i, 0)),
          pl.BlockSpec((blk, H), lambda i: (i, 0)),
          pl.BlockSpec((H, H), lambda i: (0, 0)),
          pl.BlockSpec((H, H), lambda i: (0, 0)),
          pl.BlockSpec((1, H), lambda i: (0, 0)),
      ],
      out_specs=[
          pl.BlockSpec((blk, H), lambda i: (i, 0)),
          pl.BlockSpec((blk, H), lambda i: (i, 0)),
      ],
      out_shape=[jax.ShapeDtypeStruct((NM, H), F32),
                 jax.ShapeDtypeStruct((NG, H), F32)],
  )(mesh_nodes, grid_nodes, w1b, w1c, be1)


def _make_gather(geo):
  epw, chg, kg, nsupg = geo["epw"], geo["chg"], geo["kg"], geo["nsupg"]
  srg = kg * chg
  nchunkg = epw // chg
  neh = epw * NW

  def body(psrc_hbm, pdst_hbm, srcr_hbm, dstr_hbm, gs_hbm,
           idxs_v, idxd_v, rows_v, sg0, sg1, sw0, sw1):
    c = lax.axis_index("c")
    s = lax.axis_index("s")
    wid = c * NS + s
    ebase = wid * epw
    pltpu.sync_copy(srcr_hbm.at[wid], idxs_v)
    pltpu.sync_copy(dstr_hbm.at[wid], idxd_v)
    sg = (sg0, sg1)
    sw = (sw0, sw1)

    def fire_src(m, p):
      for k in range(kg):
        pltpu.async_copy(psrc_hbm.at[idxs_v.at[m * kg + k]],
                         rows_v.at[p, pl.ds(k * chg, chg)], sg[p])

    def fire_add(m, p):
      for k in range(kg):
        pltpu.async_copy(pdst_hbm.at[idxd_v.at[m * kg + k]],
                         rows_v.at[p, pl.ds(k * chg, chg)], sg[p],
                         add=True)

    def drain_g(p):
      for k in range(kg):
        pltpu.make_async_copy(psrc_hbm.at[idxs_v.at[k]],
                              rows_v.at[p, pl.ds(k * chg, chg)],
                              sg[p]).wait()

    def wait_w(p):
      pltpu.make_async_copy(rows_v.at[p],
                            gs_hbm.at[pl.ds(ebase, srg)], sw[p]).wait()

    def mid(m, p, q):
      drain_g(p)
      fire_add(m, p)
      @pl.when(m >= 1)
      def _():
        wait_w(q)

      @pl.when(m + 1 < nsupg)
      def _():
        fire_src(m + 1, q)
      drain_g(p)
      pltpu.async_copy(rows_v.at[p],
                       gs_hbm.at[pl.ds(ebase + m * srg, srg)], sw[p])

    fire_src(0, 0)

    def step(mm, carry):
      for p in range(2):
        mid(2 * mm + p, p, 1 - p)
      return carry
    lax.fori_loop(0, nsupg // 2, step, 0)
    if nsupg % 2:
      mid(jnp.int32(nsupg - 1), 0, 1)
      wait_w(0)
    else:
      wait_w(1)

  return pl.kernel(
      body,
      out_type=jax.ShapeDtypeStruct((neh, H), F32),
      mesh=_mesh(),
      scratch_types=[
          pltpu.VMEM((nchunkg, chg), jnp.int32),
          pltpu.VMEM((nchunkg, chg), jnp.int32),
          pltpu.VMEM((2, srg, H), F32),
          pltpu.SemaphoreType.DMA,
          pltpu.SemaphoreType.DMA,
          pltpu.SemaphoreType.DMA,
          pltpu.SemaphoreType.DMA,
      ],
  )


def _edge_body(e_ref, gs_ref, w1a_ref, w2_ref, be2_ref,
               gam_ref, bet_ref, out_ref):
  e = e_ref[...]
  h = jnp.dot(e.astype(BF16), w1a_ref[...], preferred_element_type=F32)
  h = h + gs_ref[...]
  h = h * jax.nn.sigmoid(h)
  h = jnp.dot(h.astype(BF16), w2_ref[...],
              preferred_element_type=F32) + be2_ref[...]
  mu = jnp.mean(h, axis=-1, keepdims=True)
  ms = jnp.mean(h * h, axis=-1, keepdims=True)
  out_ref[...] = ((h - mu) * lax.rsqrt(ms - mu * mu + 1e-5) * gam_ref[...]
                  + bet_ref[...] + e)


def _edge_mlp(e_feats, off, neh, gs, w1a, w2, be2, gam, bet):
  base = off // EBLK
  return pl.pallas_call(
      _edge_body,
      grid=(neh // EBLK,),
      in_specs=[
          pl.BlockSpec((EBLK, H), lambda i: (i + base, 0)),
          pl.BlockSpec((EBLK, H), lambda i: (i, 0)),
          pl.BlockSpec((H, H), lambda i: (0, 0)),
          pl.BlockSpec((H, H), lambda i: (0, 0)),
          pl.BlockSpec((1, H), lambda i: (0, 0)),
          pl.BlockSpec((1, H), lambda i: (0, 0)),
          pl.BlockSpec((1, H), lambda i: (0, 0)),
      ],
      out_specs=pl.BlockSpec((EBLK, H), lambda i: (i, 0)),
      out_shape=jax.ShapeDtypeStruct((neh, H), F32),
      compiler_params=pltpu.CompilerParams(
          dimension_semantics=("arbitrary",)),
  )(e_feats, gs, w1a, w2, be2, gam, bet)


def _make_scatter(geo):
  epw, chs, ks, nsups = geo["epw"], geo["chs"], geo["ks"], geo["nsups"]
  srs = ks * chs

  def body(ef_hbm, dstw_hbm, out_hbm, acc_sh, val_v, idx_v, sem_in):
    c = lax.axis_index("c")
    s = lax.axis_index("s")
    wid = c * NS + s
    ebase = wid * epw

    zero = jnp.zeros((16,), F32)

    def zrow(r, carry):
      for k in range(H // 16):
        val_v[r, pl.ds(16 * k, 16)] = zero
      return carry
    lax.fori_loop(0, ZC, zrow, 0)
    rbase = s * ACC_PW
    for z in range(ACC_PW // ZC):
      pltpu.sync_copy(val_v.at[pl.ds(0, ZC)],
                      acc_sh.at[pl.ds(rbase + z * ZC, ZC)])
    plsc.subcore_barrier()

    def superchunk(m, carry):
      icp = pltpu.async_copy(dstw_hbm.at[wid, m], idx_v, sem_in)
      vcp = pltpu.async_copy(ef_hbm.at[pl.ds(ebase + m * srs, srs)],
                             val_v, sem_in)
      icp.wait()
      vcp.wait()
      adds = []
      for k in range(ks):
        adds.append(pltpu.async_copy(val_v.at[pl.ds(k * chs, chs)],
                                     acc_sh.at[idx_v.at[k]], sem_in,
                                     add=True))
      for cp in adds:
        cp.wait()
      return carry
    lax.fori_loop(0, nsups, superchunk, 0)
    plsc.subcore_barrier()

    for z in range(ACC_PW // ZC):
      pltpu.sync_copy(acc_sh.at[pl.ds(rbase + z * ZC, ZC)],
                      val_v.at[pl.ds(0, ZC)])
      pltpu.sync_copy(val_v.at[pl.ds(0, ZC)],
                      out_hbm.at[c, pl.ds(rbase + z * ZC, ZC)])

  return pl.kernel(
      body,
      out_type=jax.ShapeDtypeStruct((NC, NGP, H), F32),
      mesh=_mesh(),
      scratch_types=[
          pltpu.VMEM_SHARED((NGP, H), F32),
          pltpu.VMEM((srs, H), F32),
          pltpu.VMEM((ks, chs), jnp.int32),
          pltpu.SemaphoreType.DMA,
      ],
  )


def _node_body(g_ref, p0_ref, p1_ref, p2_ref, p3_ref,
               wn1a_ref, wn1b_ref, bn1_ref,
               wn2_ref, bn2_ref, gam_ref, bet_ref, out_ref):
  g = g_ref[...]
  agg = ((p0_ref[0] + p0_ref[1]) + (p1_ref[0] + p1_ref[1])
         + (p2_ref[0] + p2_ref[1]) + (p3_ref[0] + p3_ref[1]))
  h = (jnp.dot(g.astype(BF16), wn1a_ref[...], preferred_element_type=F32)
       + jnp.dot(agg.astype(BF16), wn1b_ref[...],
                 preferred_element_type=F32)
       + bn1_ref[...])
  h = h * jax.nn.sigmoid(h)
  h = jnp.dot(h.astype(BF16), wn2_ref[...],
              preferred_element_type=F32) + bn2_ref[...]
  mu = jnp.mean(h, axis=-1, keepdims=True)
  var = jnp.mean((h - mu) * (h - mu), axis=-1, keepdims=True)
  out_ref[...] = ((h - mu) * lax.rsqrt(var + 1e-5) * gam_ref[...]
                  + bet_ref[...] + g)


def _node_mlp(grid_nodes, parts, wn1a, wn1b, bn1, wn2, bn2, gam, bet):
  blk = 2000
  return pl.pallas_call(
      _node_body,
      grid=(NG // blk,),
      in_specs=[
          pl.BlockSpec((blk, H), lambda i: (i, 0)),
          pl.BlockSpec((NC, blk, H), lambda i: (0, i, 0)),
          pl.BlockSpec((NC, blk, H), lambda i: (0, i, 0)),
          pl.BlockSpec((NC, blk, H), lambda i: (0, i, 0)),
          pl.BlockSpec((NC, blk, H), lambda i: (0, i, 0)),
          pl.BlockSpec((H, H), lambda i: (0, 0)),
          pl.BlockSpec((H, H), lambda i: (0, 0)),
          pl.BlockSpec((1, H), lambda i: (0, 0)),
          pl.BlockSpec((H, H), lambda i: (0, 0)),
          pl.BlockSpec((1, H), lambda i: (0, 0)),
          pl.BlockSpec((1, H), lambda i: (0, 0)),
          pl.BlockSpec((1, H), lambda i: (0, 0)),
      ],
      out_specs=pl.BlockSpec((blk, H), lambda i: (i, 0)),
      out_shape=jax.ShapeDtypeStruct((NG, H), F32),
  )(grid_nodes, *parts, wn1a, wn1b, bn1, wn2, bn2, gam, bet)


def kernel(mesh2grid_edge_features, grid_node_features, mesh_node_features,
           mesh2grid_edge_indices_src, mesh2grid_edge_indices_dst,
           We1, be1, We2, be2, e_gamma, e_beta,
           Wn1, bn1, Wn2, bn2, n_gamma, n_beta):
  w1a = We1[:H]
  w1b = We1[H:2 * H]
  w1c = We1[2 * H:]
  wn1a = Wn1[:H]
  wn1b = Wn1[H:]
  r1 = lambda v: v.reshape(1, H)

  src = mesh2grid_edge_indices_src
  dst = mesh2grid_edge_indices_dst

  psrc, pdst = _project(mesh_node_features, grid_node_features,
                        w1b, w1c, r1(be1))

  w1a_b = w1a.astype(BF16)
  w2_b = We2.astype(BF16)
  e_args = (w1a_b, w2_b, r1(be2), r1(e_gamma), r1(e_beta))

  gather_fns = {id(GEO_A): _make_gather(GEO_A), id(GEO_B): _make_gather(GEO_B)}
  scatter_fns = {id(GEO_A): _make_scatter(GEO_A),
                 id(GEO_B): _make_scatter(GEO_B)}

  efs, dws, geos = [], [], []
  for off, geo in SLICES:
    epw, chg, kg = geo["epw"], geo["chg"], geo["kg"]
    chs, ks = geo["chs"], geo["ks"]
    neh = epw * NW
    end = off + neh
    nchunkg = epw // chg
    src_r = src[off:end].reshape(NW, nchunkg, chg)
    dst_r = dst[off:end].reshape(NW, nchunkg, chg)
    dws.append(dst[off:end].reshape(NW, epw // (ks * chs), ks, chs))
    geos.append(geo)
    g = gather_fns[id(geo)](psrc, pdst, src_r, dst_r)
    efs.append(_edge_mlp(mesh2grid_edge_features, off, neh, g, *e_args))

  parts = [scatter_fns[id(geo)](ef, dw)
           for ef, dw, geo in zip(efs, dws, geos)]

  out = _node_mlp(grid_node_features, parts,
                  wn1a.astype(BF16), wn1b.astype(BF16), r1(bn1),
                  Wn2.astype(BF16), r1(bn2), r1(n_gamma), r1(n_beta))
  return out
